# baseline (speedup 1.0000x reference)
import functools

import jax
import jax.numpy as jnp
from jax import lax
from jax.experimental import pallas as pl
from jax.experimental.pallas import tpu as pltpu
from jax.experimental.pallas import tpu_sc as plsc

NC = 2
NS = 16
K = 120
NBUF = 3
M_FAST = 162
M_SLOW = 6
FAST_CORE = 0


def _sc_scatter(x2d, idx5, zeros2d, m_fast, m_slow, fast_core, acc_rows, d):
    mesh = plsc.VectorSubcoreMesh(core_axis_name="c", subcore_axis_name="s")
    zero_rows = acc_rows // NS
    out_rows = acc_rows // NS
    nbuf = NBUF
    assert m_fast % (2 * nbuf) == 0 and m_slow % (2 * nbuf) == 0
    assert m_slow >= 2 * nbuf

    @functools.partial(
        pl.kernel,
        out_type=jax.ShapeDtypeStruct((NC, acc_rows, d), jnp.float32),
        mesh=mesh,
        scratch_types=(
            [pltpu.VMEM((K, d), jnp.float32) for _ in range(nbuf)]
            + [pltpu.VMEM((2, K), jnp.int32) for _ in range(2 * nbuf)]
            + [pltpu.VMEM_SHARED((acc_rows, d), jnp.float32)]
            + [pltpu.SemaphoreType.DMA for _ in range(4 * nbuf)]
        ),
    )
    def k(x_hbm, idx_hbm, z_hbm, part_hbm, *rest):
        bufs = rest[:nbuf]
        idxb = rest[nbuf:3 * nbuf]
        acc = rest[3 * nbuf]
        gsem = rest[3 * nbuf + 1:4 * nbuf + 1]
        ssem = rest[4 * nbuf + 1:5 * nbuf + 1]
        isem = rest[5 * nbuf + 1:]
        c = lax.axis_index("c")
        s = lax.axis_index("s")
        m_c = jnp.where(c == fast_core, m_fast, m_slow)

        zb = s * zero_rows
        pltpu.sync_copy(z_hbm.at[pl.ds(zb, zero_rows)],
                        acc.at[pl.ds(zb, zero_rows)])
        plsc.subcore_barrier()

        for ib in range(2 * nbuf):
            pltpu.async_copy(idx_hbm.at[c, s, ib], idxb[ib], isem[ib])
        for b in range(nbuf):
            pltpu.make_async_copy(
                idx_hbm.at[c, s, b], idxb[b], isem[b]).wait()
            pltpu.async_copy(x_hbm.at[idxb[b].at[0]], bufs[b], gsem[b])

        def body(i2, carry):
            j0 = i2 * 2 * nbuf
            for bb in range(2 * nbuf):
                b = bb % nbuf
                ibn = (bb + nbuf) % (2 * nbuf)
                j = j0 + bb
                pltpu.make_async_copy(
                    x_hbm.at[idxb[bb].at[0]], bufs[b], gsem[b]).wait()
                pltpu.async_copy(
                    bufs[b], acc.at[idxb[bb].at[1]], ssem[b], add=True)
                pltpu.make_async_copy(
                    bufs[b], acc.at[idxb[bb].at[1]], ssem[b]).wait()
                jn2 = jnp.minimum(j + 2 * nbuf, m_c - 1)
                pltpu.async_copy(idx_hbm.at[c, s, jn2], idxb[bb], isem[bb])
                pltpu.make_async_copy(
                    idx_hbm.at[c, s, j], idxb[ibn], isem[ibn]).wait()
                pltpu.async_copy(x_hbm.at[idxb[ibn].at[0]], bufs[b], gsem[b])
            return carry

        lax.fori_loop(0, m_c // (2 * nbuf), body, 0)
        for b in range(nbuf):
            pltpu.make_async_copy(
                x_hbm.at[idxb[b].at[0]], bufs[b], gsem[b]).wait()
        for ib in range(nbuf, 2 * nbuf):
            pltpu.make_async_copy(
                idx_hbm.at[c, s, 0], idxb[ib], isem[ib]).wait()
        plsc.subcore_barrier()
        ob = s * out_rows
        pltpu.sync_copy(acc.at[pl.ds(ob, out_rows)],
                        part_hbm.at[c, pl.ds(ob, out_rows)])

    return k(x2d, idx5, zeros2d)


def _tc_combine(part, weight, bias, n, d):
    blk = 400

    def body(p_ref, w_ref, b_ref, o_ref):
        p = p_ref[0] + p_ref[1]
        o_ref[...] = (
            jnp.dot(p, w_ref[...], preferred_element_type=jnp.float32)
            + b_ref[...]
        )

    return pl.pallas_call(
        body,
        grid=(n // blk,),
        in_specs=[
            pl.BlockSpec((NC, blk, d), lambda i: (0, i, 0)),
            pl.BlockSpec((d, d), lambda i: (0, 0)),
            pl.BlockSpec((1, d), lambda i: (0, 0)),
        ],
        out_specs=pl.BlockSpec((blk, d), lambda i: (i, 0)),
        out_shape=jax.ShapeDtypeStruct((n, d), jnp.float32),
    )(part, weight, bias.reshape(1, d))


def kernel(x, edge_index, weight, bias):
    b, n, d_in = x.shape
    d_out = weight.shape[1]
    x2d = x.reshape(n, d_in)
    ei = edge_index.astype(jnp.int32)
    row, col = ei[0], ei[1]
    e = row.shape[0]

    cap_fast = NS * M_FAST * K
    cap_slow = NS * M_SLOW * K
    e_pad = cap_fast + cap_slow
    assert e_pad >= e, (e_pad, e)
    pad = e_pad - e
    row_p = jnp.concatenate([row, jnp.zeros((pad,), jnp.int32)])
    col_p = jnp.concatenate([col, jnp.full((pad,), n, jnp.int32)])

    def grid(rp, cp, m):
        return jnp.stack([rp.reshape(NS, m, K), cp.reshape(NS, m, K)],
                         axis=2)

    g_fast = grid(row_p[:cap_fast], col_p[:cap_fast], M_FAST)
    g_slow = jnp.concatenate([
        grid(row_p[cap_fast:], col_p[cap_fast:], M_SLOW),
        jnp.zeros((NS, M_FAST - M_SLOW, 2, K), jnp.int32),
    ], axis=1)
    cores = [g_fast, g_slow] if FAST_CORE == 0 else [g_slow, g_fast]
    idx5 = jnp.stack(cores, axis=0)

    acc_rows = -(-(n + 1) // (NS * 8)) * (NS * 8)
    zeros2d = jnp.zeros((acc_rows, d_in), jnp.float32)

    part = _sc_scatter(x2d, idx5, zeros2d, M_FAST, M_SLOW, FAST_CORE,
                       acc_rows, d_in)
    assert n % 400 == 0
    out = _tc_combine(part, weight, bias, n, d_out)
    return out.reshape(b, n, d_out)


# speedup vs baseline: 1.1279x; 1.1279x over previous
import functools

import jax
import jax.numpy as jnp
from jax import lax
from jax.experimental import pallas as pl
from jax.experimental.pallas import tpu as pltpu
from jax.experimental.pallas import tpu_sc as plsc

NC = 2
NS = 16
K = 120
NBUF = 3
M_FAST = 144
M_SLOW = 24
FAST_CORE = 0


def _sc_scatter(x2d, idx5, zeros2d, m_fast, m_slow, fast_core, acc_rows, d):
    mesh = plsc.VectorSubcoreMesh(core_axis_name="c", subcore_axis_name="s")
    zero_rows = acc_rows // NS
    out_rows = acc_rows // NS
    nbuf = NBUF
    assert m_fast % (2 * nbuf) == 0 and m_slow % (2 * nbuf) == 0
    assert m_slow >= 2 * nbuf

    @functools.partial(
        pl.kernel,
        out_type=jax.ShapeDtypeStruct((NC, acc_rows, d), jnp.float32),
        mesh=mesh,
        scratch_types=(
            [pltpu.VMEM((K, d), jnp.float32) for _ in range(nbuf)]
            + [pltpu.VMEM((2, K), jnp.int32) for _ in range(2 * nbuf)]
            + [pltpu.VMEM_SHARED((acc_rows, d), jnp.float32)]
            + [pltpu.SemaphoreType.DMA for _ in range(4 * nbuf)]
        ),
    )
    def k(x_hbm, idx_hbm, z_hbm, part_hbm, *rest):
        bufs = rest[:nbuf]
        idxb = rest[nbuf:3 * nbuf]
        acc = rest[3 * nbuf]
        gsem = rest[3 * nbuf + 1:4 * nbuf + 1]
        ssem = rest[4 * nbuf + 1:5 * nbuf + 1]
        isem = rest[5 * nbuf + 1:]
        c = lax.axis_index("c")
        s = lax.axis_index("s")
        m_c = jnp.where(c == fast_core, m_fast, m_slow)

        zb = s * zero_rows
        pltpu.sync_copy(z_hbm.at[pl.ds(zb, zero_rows)],
                        acc.at[pl.ds(zb, zero_rows)])
        plsc.subcore_barrier()

        for ib in range(2 * nbuf):
            pltpu.async_copy(idx_hbm.at[c, s, ib], idxb[ib], isem[ib])
        for b in range(nbuf):
            pltpu.make_async_copy(
                idx_hbm.at[c, s, b], idxb[b], isem[b]).wait()
            pltpu.async_copy(x_hbm.at[idxb[b].at[0]], bufs[b], gsem[b])

        def body(i2, carry):
            j0 = i2 * 2 * nbuf
            for bb in range(2 * nbuf):
                b = bb % nbuf
                ibn = (bb + nbuf) % (2 * nbuf)
                j = j0 + bb
                pltpu.make_async_copy(
                    x_hbm.at[idxb[bb].at[0]], bufs[b], gsem[b]).wait()
                pltpu.async_copy(
                    bufs[b], acc.at[idxb[bb].at[1]], ssem[b], add=True)
                pltpu.make_async_copy(
                    bufs[b], acc.at[idxb[bb].at[1]], ssem[b]).wait()
                jn2 = jnp.minimum(j + 2 * nbuf, m_c - 1)
                pltpu.async_copy(idx_hbm.at[c, s, jn2], idxb[bb], isem[bb])
                pltpu.make_async_copy(
                    idx_hbm.at[c, s, j], idxb[ibn], isem[ibn]).wait()
                pltpu.async_copy(x_hbm.at[idxb[ibn].at[0]], bufs[b], gsem[b])
            return carry

        lax.fori_loop(0, m_c // (2 * nbuf), body, 0)
        for b in range(nbuf):
            pltpu.make_async_copy(
                x_hbm.at[idxb[b].at[0]], bufs[b], gsem[b]).wait()
        for ib in range(nbuf, 2 * nbuf):
            pltpu.make_async_copy(
                idx_hbm.at[c, s, 0], idxb[ib], isem[ib]).wait()
        plsc.subcore_barrier()
        ob = s * out_rows
        pltpu.sync_copy(acc.at[pl.ds(ob, out_rows)],
                        part_hbm.at[c, pl.ds(ob, out_rows)])

    return k(x2d, idx5, zeros2d)


def _tc_combine(part, weight, bias, n, d):
    blk = 400

    def body(p_ref, w_ref, b_ref, o_ref):
        p = p_ref[0] + p_ref[1]
        o_ref[...] = (
            jnp.dot(p, w_ref[...], preferred_element_type=jnp.float32)
            + b_ref[...]
        )

    return pl.pallas_call(
        body,
        grid=(n // blk,),
        in_specs=[
            pl.BlockSpec((NC, blk, d), lambda i: (0, i, 0)),
            pl.BlockSpec((d, d), lambda i: (0, 0)),
            pl.BlockSpec((1, d), lambda i: (0, 0)),
        ],
        out_specs=pl.BlockSpec((blk, d), lambda i: (i, 0)),
        out_shape=jax.ShapeDtypeStruct((n, d), jnp.float32),
    )(part, weight, bias.reshape(1, d))


def kernel(x, edge_index, weight, bias):
    b, n, d_in = x.shape
    d_out = weight.shape[1]
    x2d = x.reshape(n, d_in)
    ei = edge_index.astype(jnp.int32)
    row, col = ei[0], ei[1]
    e = row.shape[0]

    cap_fast = NS * M_FAST * K
    cap_slow = NS * M_SLOW * K
    e_pad = cap_fast + cap_slow
    assert e_pad >= e, (e_pad, e)
    pad = e_pad - e
    row_p = jnp.concatenate([row, jnp.zeros((pad,), jnp.int32)])
    col_p = jnp.concatenate([col, jnp.full((pad,), n, jnp.int32)])

    def grid(rp, cp, m):
        return jnp.stack([rp.reshape(NS, m, K), cp.reshape(NS, m, K)],
                         axis=2)

    g_fast = grid(row_p[:cap_fast], col_p[:cap_fast], M_FAST)
    g_slow = jnp.concatenate([
        grid(row_p[cap_fast:], col_p[cap_fast:], M_SLOW),
        jnp.zeros((NS, M_FAST - M_SLOW, 2, K), jnp.int32),
    ], axis=1)
    cores = [g_fast, g_slow] if FAST_CORE == 0 else [g_slow, g_fast]
    idx5 = jnp.stack(cores, axis=0)

    acc_rows = -(-(n + 1) // (NS * 8)) * (NS * 8)
    zeros2d = jnp.zeros((acc_rows, d_in), jnp.float32)

    part = _sc_scatter(x2d, idx5, zeros2d, M_FAST, M_SLOW, FAST_CORE,
                       acc_rows, d_in)
    assert n % 400 == 0
    out = _tc_combine(part, weight, bias, n, d_out)
    return out.reshape(b, n, d_out)
